# baseline (speedup 1.0000x reference)
import functools

import jax
import jax.numpy as jnp
from jax import lax
from jax.experimental import pallas as pl
from jax.experimental.pallas import tpu as pltpu
from jax.experimental.pallas import tpu_sc as plsc

N_EMB = 8192
DIM = 256
COMMIT = 0.25

BM = 256
BN = 1024


def _argmin_body(x_ref, e_ref, sx_ref, se_ref, idx_ref, loss_ref,
                 min_sc, idx_sc):
    j = pl.program_id(1)
    n_j = pl.num_programs(1)

    @pl.when(j == 0)
    def _init():
        min_sc[...] = jnp.full((BM, 1), jnp.inf, jnp.float32)
        idx_sc[...] = jnp.zeros((BM, 1), jnp.int32)

    mm = lax.dot_general(
        x_ref[...].astype(jnp.bfloat16), e_ref[...].astype(jnp.bfloat16),
        dimension_numbers=(((1,), (1,)), ((), ())),
        preferred_element_type=jnp.float32)
    d = (sx_ref[...] + se_ref[...]) - 2.0 * mm

    tmin = jnp.min(d, axis=1, keepdims=True)
    col = lax.broadcasted_iota(jnp.int32, (BM, BN), 1) + j * BN
    cand = jnp.where(d == tmin, col, N_EMB)
    targ = jnp.min(cand, axis=1, keepdims=True)

    better = tmin < min_sc[...]
    min_sc[...] = jnp.where(better, tmin, min_sc[...])
    idx_sc[...] = jnp.where(better, targ, idx_sc[...])

    @pl.when(j == n_j - 1)
    def _fin():
        idx_ref[...] = idx_sc[...]
        loss_ref[...] = jnp.full((1, 1, 128), jnp.sum(min_sc[...]), jnp.float32)


def _argmin_call(flat_x, emb, sx, se):
    n, m = flat_x.shape[0], emb.shape[0]
    grid = (n // BM, m // BN)
    return pl.pallas_call(
        _argmin_body,
        grid=grid,
        in_specs=[
            pl.BlockSpec((BM, DIM), lambda i, j: (i, 0)),
            pl.BlockSpec((BN, DIM), lambda i, j: (j, 0)),
            pl.BlockSpec((BM, 1), lambda i, j: (i, 0)),
            pl.BlockSpec((1, BN), lambda i, j: (0, j)),
        ],
        out_specs=[
            pl.BlockSpec((BM, 1), lambda i, j: (i, 0)),
            pl.BlockSpec((1, 1, 128), lambda i, j: (i, 0, 0)),
        ],
        out_shape=[
            jax.ShapeDtypeStruct((n, 1), jnp.int32),
            jax.ShapeDtypeStruct((grid[0], 1, 128), jnp.float32),
        ],
        scratch_shapes=[
            pltpu.VMEM((BM, 1), jnp.float32),
            pltpu.VMEM((BM, 1), jnp.int32),
        ],
    )(flat_x, emb, sx, se)


def _make_sc_gather(n_rows):
    info = plsc.get_sparse_core_info()
    nc, ns = info.num_cores, info.num_subcores
    nw = nc * ns
    b_per_w = n_rows // nw
    chunk = 128
    n_ch = b_per_w // chunk
    mesh = plsc.VectorSubcoreMesh(core_axis_name="c", subcore_axis_name="s")

    @functools.partial(
        pl.kernel, mesh=mesh,
        out_type=jax.ShapeDtypeStruct((n_rows, DIM), jnp.float32),
        scratch_types=[
            pltpu.VMEM((chunk,), jnp.int32),
            pltpu.VMEM((chunk, DIM), jnp.float32),
            pltpu.SemaphoreType.DMA,
        ],
    )
    def gather_k(table_hbm, idx_hbm, out_hbm, idx_v, rows_v, sem):
        wid = lax.axis_index("s") * nc + lax.axis_index("c")
        base = wid * b_per_w
        for c in range(n_ch):
            off = base + c * chunk
            pltpu.sync_copy(idx_hbm.at[pl.ds(off, chunk)], idx_v)
            pltpu.async_copy(table_hbm.at[idx_v], rows_v, sem).wait()
            pltpu.sync_copy(rows_v, out_hbm.at[pl.ds(off, chunk)])

    return gather_k


def kernel(inputs, embedding):
    b, c, h, w = inputs.shape
    n = b * h * w
    flat_x = jnp.transpose(inputs, (0, 2, 3, 1)).reshape(-1, DIM)
    sx = jnp.sum(flat_x ** 2, axis=1, keepdims=True)
    se = jnp.sum(embedding ** 2, axis=1).reshape(1, -1)

    idx2d, loss_parts = _argmin_call(flat_x, embedding, sx, se)
    idx = idx2d.reshape(-1)

    q_flat = _make_sc_gather(n)(embedding, idx)

    loss = (1.0 + COMMIT) * jnp.sum(loss_parts[:, 0, 0]) / (n * DIM)
    q = jnp.transpose(q_flat.reshape(b, h, w, c), (0, 3, 1, 2))
    q_st = inputs + (q - inputs)
    return q_st, loss, idx.reshape(b, h, w)


# speedup vs baseline: 1.9315x; 1.9315x over previous
import functools

import jax
import jax.numpy as jnp
from jax import lax
from jax.experimental import pallas as pl
from jax.experimental.pallas import tpu as pltpu
from jax.experimental.pallas import tpu_sc as plsc

N_EMB = 8192
DIM = 256
COMMIT = 0.25

BM = 512
BN = 2048


def _argmin_body(x_ref, e_ref, sx_ref, se_ref, idx_ref, loss_ref,
                 min_sc, idx_sc):
    j = pl.program_id(1)
    n_j = pl.num_programs(1)

    @pl.when(j == 0)
    def _init():
        min_sc[...] = jnp.full((BM, 1), jnp.inf, jnp.float32)
        idx_sc[...] = jnp.zeros((BM, 1), jnp.int32)

    mm = lax.dot_general(
        x_ref[...].astype(jnp.bfloat16), e_ref[...].astype(jnp.bfloat16),
        dimension_numbers=(((1,), (1,)), ((), ())),
        preferred_element_type=jnp.float32)
    d = (sx_ref[...] + se_ref[...]) - 2.0 * mm

    tmin = jnp.min(d, axis=1, keepdims=True)
    col = lax.broadcasted_iota(jnp.int32, (BM, BN), 1) + j * BN
    cand = jnp.where(d == tmin, col, N_EMB)
    targ = jnp.min(cand, axis=1, keepdims=True)

    better = tmin < min_sc[...]
    min_sc[...] = jnp.where(better, tmin, min_sc[...])
    idx_sc[...] = jnp.where(better, targ, idx_sc[...])

    @pl.when(j == n_j - 1)
    def _fin():
        idx_ref[...] = idx_sc[...]
        loss_ref[...] = jnp.full((1, 1, 128), jnp.sum(min_sc[...]), jnp.float32)


def _argmin_call(flat_x, emb, sx, se):
    n, m = flat_x.shape[0], emb.shape[0]
    grid = (n // BM, m // BN)
    return pl.pallas_call(
        _argmin_body,
        grid=grid,
        in_specs=[
            pl.BlockSpec((BM, DIM), lambda i, j: (i, 0)),
            pl.BlockSpec((BN, DIM), lambda i, j: (j, 0)),
            pl.BlockSpec((BM, 1), lambda i, j: (i, 0)),
            pl.BlockSpec((1, BN), lambda i, j: (0, j)),
        ],
        out_specs=[
            pl.BlockSpec((BM, 1), lambda i, j: (i, 0)),
            pl.BlockSpec((1, 1, 128), lambda i, j: (i, 0, 0)),
        ],
        out_shape=[
            jax.ShapeDtypeStruct((n, 1), jnp.int32),
            jax.ShapeDtypeStruct((grid[0], 1, 128), jnp.float32),
        ],
        scratch_shapes=[
            pltpu.VMEM((BM, 1), jnp.float32),
            pltpu.VMEM((BM, 1), jnp.int32),
        ],
    )(flat_x, emb, sx, se)


def _make_sc_gather(n_rows):
    info = plsc.get_sparse_core_info()
    nc, ns = info.num_cores, info.num_subcores
    nw = nc * ns
    b_per_w = n_rows // nw
    chunk = 128
    n_ch = b_per_w // chunk
    mesh = plsc.VectorSubcoreMesh(core_axis_name="c", subcore_axis_name="s")

    @functools.partial(
        pl.kernel, mesh=mesh,
        out_type=jax.ShapeDtypeStruct((n_rows, DIM), jnp.float32),
        scratch_types=[
            pltpu.VMEM((chunk,), jnp.int32),
            pltpu.VMEM((chunk, DIM), jnp.float32),
            pltpu.SemaphoreType.DMA,
        ],
    )
    def gather_k(table_hbm, idx_hbm, out_hbm, idx_v, rows_v, sem):
        wid = lax.axis_index("s") * nc + lax.axis_index("c")
        base = wid * b_per_w
        for c in range(n_ch):
            off = base + c * chunk
            pltpu.sync_copy(idx_hbm.at[pl.ds(off, chunk)], idx_v)
            pltpu.async_copy(table_hbm.at[idx_v], rows_v, sem).wait()
            pltpu.sync_copy(rows_v, out_hbm.at[pl.ds(off, chunk)])

    return gather_k


def kernel(inputs, embedding):
    b, c, h, w = inputs.shape
    n = b * h * w
    flat_x = jnp.transpose(inputs, (0, 2, 3, 1)).reshape(-1, DIM)
    sx = jnp.sum(flat_x ** 2, axis=1, keepdims=True)
    se = jnp.sum(embedding ** 2, axis=1).reshape(1, -1)

    idx2d, loss_parts = _argmin_call(flat_x, embedding, sx, se)
    idx = idx2d.reshape(-1)

    q_flat = _make_sc_gather(n)(embedding, idx)

    loss = (1.0 + COMMIT) * jnp.sum(loss_parts[:, 0, 0]) / (n * DIM)
    q = jnp.transpose(q_flat.reshape(b, h, w, c), (0, 3, 1, 2))
    q_st = inputs + (q - inputs)
    return q_st, loss, idx.reshape(b, h, w)


# speedup vs baseline: 2.1983x; 1.1381x over previous
import functools

import jax
import jax.numpy as jnp
from jax import lax
from jax.experimental import pallas as pl
from jax.experimental.pallas import tpu as pltpu
from jax.experimental.pallas import tpu_sc as plsc

N_EMB = 8192
DIM = 256
COMMIT = 0.25

BM = 1024
BN = 2048


def _argmin_body(x_ref, e_ref, sx_ref, se_ref, idx_ref, loss_ref,
                 min_sc, idx_sc):
    j = pl.program_id(1)
    n_j = pl.num_programs(1)

    @pl.when(j == 0)
    def _init():
        min_sc[...] = jnp.full((BM, 1), jnp.inf, jnp.float32)
        idx_sc[...] = jnp.zeros((BM, 1), jnp.int32)

    mm = lax.dot_general(
        x_ref[...].astype(jnp.bfloat16), e_ref[...].astype(jnp.bfloat16),
        dimension_numbers=(((1,), (1,)), ((), ())),
        preferred_element_type=jnp.float32)
    d = (sx_ref[...] + se_ref[...]) - 2.0 * mm

    tmin = jnp.min(d, axis=1, keepdims=True)
    col = lax.broadcasted_iota(jnp.int32, (BM, BN), 1) + j * BN
    cand = jnp.where(d == tmin, col, N_EMB)
    targ = jnp.min(cand, axis=1, keepdims=True)

    better = tmin < min_sc[...]
    min_sc[...] = jnp.where(better, tmin, min_sc[...])
    idx_sc[...] = jnp.where(better, targ, idx_sc[...])

    @pl.when(j == n_j - 1)
    def _fin():
        idx_ref[...] = idx_sc[...]
        loss_ref[...] = jnp.full((1, 1, 128), jnp.sum(min_sc[...]), jnp.float32)


def _argmin_call(flat_x, emb, sx, se):
    n, m = flat_x.shape[0], emb.shape[0]
    grid = (n // BM, m // BN)
    return pl.pallas_call(
        _argmin_body,
        grid=grid,
        in_specs=[
            pl.BlockSpec((BM, DIM), lambda i, j: (i, 0)),
            pl.BlockSpec((BN, DIM), lambda i, j: (j, 0)),
            pl.BlockSpec((BM, 1), lambda i, j: (i, 0)),
            pl.BlockSpec((1, BN), lambda i, j: (0, j)),
        ],
        out_specs=[
            pl.BlockSpec((BM, 1), lambda i, j: (i, 0)),
            pl.BlockSpec((1, 1, 128), lambda i, j: (i, 0, 0)),
        ],
        out_shape=[
            jax.ShapeDtypeStruct((n, 1), jnp.int32),
            jax.ShapeDtypeStruct((grid[0], 1, 128), jnp.float32),
        ],
        scratch_shapes=[
            pltpu.VMEM((BM, 1), jnp.float32),
            pltpu.VMEM((BM, 1), jnp.int32),
        ],
    )(flat_x, emb, sx, se)


def _make_sc_gather(n_rows):
    info = plsc.get_sparse_core_info()
    nc, ns = info.num_cores, info.num_subcores
    nw = nc * ns
    b_per_w = n_rows // nw
    chunk = 128
    n_ch = b_per_w // chunk
    mesh = plsc.VectorSubcoreMesh(core_axis_name="c", subcore_axis_name="s")

    @functools.partial(
        pl.kernel, mesh=mesh,
        out_type=jax.ShapeDtypeStruct((n_rows, DIM), jnp.float32),
        scratch_types=[
            pltpu.VMEM((chunk,), jnp.int32),
            pltpu.VMEM((chunk, DIM), jnp.float32),
            pltpu.SemaphoreType.DMA,
        ],
    )
    def gather_k(table_hbm, idx_hbm, out_hbm, idx_v, rows_v, sem):
        wid = lax.axis_index("s") * nc + lax.axis_index("c")
        base = wid * b_per_w
        for c in range(n_ch):
            off = base + c * chunk
            pltpu.sync_copy(idx_hbm.at[pl.ds(off, chunk)], idx_v)
            pltpu.async_copy(table_hbm.at[idx_v], rows_v, sem).wait()
            pltpu.sync_copy(rows_v, out_hbm.at[pl.ds(off, chunk)])

    return gather_k


def kernel(inputs, embedding):
    b, c, h, w = inputs.shape
    n = b * h * w
    flat_x = jnp.transpose(inputs, (0, 2, 3, 1)).reshape(-1, DIM)
    sx = jnp.sum(flat_x ** 2, axis=1, keepdims=True)
    se = jnp.sum(embedding ** 2, axis=1).reshape(1, -1)

    idx2d, loss_parts = _argmin_call(flat_x, embedding, sx, se)
    idx = idx2d.reshape(-1)

    q_flat = _make_sc_gather(n)(embedding, idx)

    loss = (1.0 + COMMIT) * jnp.sum(loss_parts[:, 0, 0]) / (n * DIM)
    q = jnp.transpose(q_flat.reshape(b, h, w, c), (0, 3, 1, 2))
    q_st = inputs + (q - inputs)
    return q_st, loss, idx.reshape(b, h, w)
